# baseline (speedup 1.0000x reference)
import functools

import jax
import jax.numpy as jnp
from jax import lax
from jax.experimental import pallas as pl
from jax.experimental.pallas import tpu as pltpu
from jax.experimental.pallas import tpu_sc as plsc

_NT = 10000
_NG = 100
_T = 8
_TS = 6
_FH = 2
_DH = 32
_DIN = 128
_DOUT = 128
_E = 160000
_W = 256
_HW = 128

_K = 128
_EPAD = 163840
_EPT = _EPAD // 16
_NCH = _EPT // _K
_NCH2 = _NCH // 2
_ACC_ROWS = _NT + 16
_RPT = 632
_RPT_LAST = _ACC_ROWS - 15 * _RPT


def _full(shape):
    nd = len(shape)
    return pl.BlockSpec(shape, lambda i, _nd=nd: (0,) * _nd)


def _cat_half(pieces, pad):
    if pad is not None:
        pieces = pieces + [pad]
    return jnp.concatenate(pieces, axis=1)


def _enc3_body(x_ref, w1, b1, w2, b2, w3, b3, wl, wr, bs, padb, u1w, u1b,
               u2w, u2b, table_ref, base_ref, dec_ref):
    B = x_ref.shape[0]
    x = x_ref[...].reshape(B * _T, _DIN)
    h = jnp.maximum(x @ w1[...] + b1[...], 0.0)
    h = jnp.maximum(h @ w2[...] + b2[...], 0.0)
    h = jnp.maximum(h @ w3[...] + b3[...], 0.0)
    h3 = h.reshape(B, _T, _DH)
    pad = jnp.broadcast_to(padb[...], (B, 64))
    zpad = jnp.zeros((B, 64), jnp.float32)
    h6 = h3[:, :_TS, :].reshape(B * _TS, _DH)
    tl = (h6 @ wl[...]).reshape(B, _TS, _DH)
    tk = [tl[:, k, :] for k in range(_TS)]
    table_ref[0] = _cat_half(tk[:4], None)
    table_ref[1] = _cat_half(tk[4:], pad)
    if base_ref is not None:
        br = (h6 @ wr[...] + bs[...]).reshape(B, _TS, _DH)
        bk = [br[:, k, :] for k in range(_TS)]
        base_ref[0] = _cat_half(bk[:4], None)
        base_ref[1] = _cat_half(bk[4:], zpad)
    d = h3[:, _TS:, :].reshape(B * _FH, _DH)
    d = jnp.maximum(d @ u1w[...] + u1b[...], 0.0)
    d = jnp.maximum(d @ u2w[...] + u2b[...], 0.0)
    dec_ref[...] = d.reshape(B, _FH, _DOUT)


def _enc3_nobase_body(x_ref, w1, b1, w2, b2, w3, b3, wl, padb, u1w, u1b,
                      u2w, u2b, table_ref, dec_ref):
    _enc3_body(x_ref, w1, b1, w2, b2, w3, b3, wl, None, None, padb, u1w, u1b,
               u2w, u2b, table_ref, None, dec_ref)


def _halves_spec(B):
    return pl.BlockSpec((2, B, _HW), lambda i: (0, i, 0))


def _pad_one():
    return jnp.zeros((1, 64), jnp.float32).at[0, 0].set(1.0)


def _encoder_t(x, p):
    B = 2000
    grid = _NT // B
    in_specs = [pl.BlockSpec((B, _T, _DIN), lambda i: (i, 0, 0)),
                _full((_DIN, _DH)), _full((1, _DH)),
                _full((_DH, _DH)), _full((1, _DH)),
                _full((_DH, _DH)), _full((1, _DH)),
                _full((_DH, _DH)), _full((_DH, _DH)), _full((1, _DH)),
                _full((1, 64)),
                _full((_DH, _DOUT)), _full((1, _DOUT)),
                _full((_DOUT, _DOUT)), _full((1, _DOUT))]
    out_specs = [_halves_spec(B), _halves_spec(B),
                 pl.BlockSpec((B, _FH, _DOUT), lambda i: (i, 0, 0))]
    out_shape = [jax.ShapeDtypeStruct((2, _NT, _HW), jnp.float32),
                 jax.ShapeDtypeStruct((2, _NT, _HW), jnp.float32),
                 jax.ShapeDtypeStruct((_NT, _FH, _DOUT), jnp.float32)]
    wr_sum = p['l0tt_Wr'] + p['l0ct_Wr'] + p['l0gt_Wr']
    b_sum = (p['l0tt_b'] + p['l0ct_b'] + p['l0gt_b']).reshape(1, _DH)
    return pl.pallas_call(
        _enc3_body, grid=(grid,), in_specs=in_specs, out_specs=out_specs,
        out_shape=out_shape)(
            x, p['t1_W'], p['t1_b'].reshape(1, _DH),
            p['t2_W'], p['t2_b'].reshape(1, _DH),
            p['t3_W'], p['t3_b'].reshape(1, _DH),
            p['l0tt_Wl'], wr_sum, b_sum, _pad_one(),
            p['up_t1_W'], p['up_t1_b'].reshape(1, _DOUT),
            p['up_t2_W'], p['up_t2_b'].reshape(1, _DOUT))


def _encoder_c(x, p):
    B = 2000
    grid = _NT // B
    in_specs = [pl.BlockSpec((B, _T, _DIN), lambda i: (i, 0, 0)),
                _full((_DIN, _DH)), _full((1, _DH)),
                _full((_DH, _DH)), _full((1, _DH)),
                _full((_DH, _DH)), _full((1, _DH)),
                _full((_DH, _DH)),
                _full((1, 64)),
                _full((_DH, _DOUT)), _full((1, _DOUT)),
                _full((_DOUT, _DOUT)), _full((1, _DOUT))]
    out_specs = [_halves_spec(B),
                 pl.BlockSpec((B, _FH, _DOUT), lambda i: (i, 0, 0))]
    out_shape = [jax.ShapeDtypeStruct((2, _NT, _HW), jnp.float32),
                 jax.ShapeDtypeStruct((_NT, _FH, _DOUT), jnp.float32)]
    return pl.pallas_call(
        _enc3_nobase_body, grid=(grid,), in_specs=in_specs,
        out_specs=out_specs, out_shape=out_shape)(
            x, p['c1_W'], p['c1_b'].reshape(1, _DH),
            p['c2_W'], p['c2_b'].reshape(1, _DH),
            p['c3_W'], p['c3_b'].reshape(1, _DH),
            p['l0ct_Wl'], _pad_one(),
            p['up_c1_W'], p['up_c1_b'].reshape(1, _DOUT),
            p['up_c2_W'], p['up_c2_b'].reshape(1, _DOUT))


def _enc_g_body(x_ref, w1, b1, w2, b2, wl, padb, u1w, u1b, u2w, u2b,
                table_ref, dec_ref):
    B = x_ref.shape[0]
    x = x_ref[...].reshape(B * _T, _DIN)
    h = jnp.maximum(x @ w1[...] + b1[...], 0.0)
    h = jnp.maximum(h @ w2[...] + b2[...], 0.0)
    h3 = h.reshape(B, _T, _DH)
    pad = jnp.broadcast_to(padb[...], (B, 64))
    h6 = h3[:, :_TS, :].reshape(B * _TS, _DH)
    tl = (h6 @ wl[...]).reshape(B, _TS, _DH)
    tk = [tl[:, k, :] for k in range(_TS)]
    table_ref[0] = _cat_half(tk[:4], None)
    table_ref[1] = _cat_half(tk[4:], pad)
    d = h3[:, _T - 1, :]
    d = jnp.maximum(d @ u1w[...] + u1b[...], 0.0)
    d = jnp.maximum(d @ u2w[...] + u2b[...], 0.0)
    dec_ref[...] = d


def _encoder_g(x, p):
    in_specs = [pl.BlockSpec((_NG, _T, _DIN), lambda i: (0, 0, 0)),
                _full((_DIN, _DH)), _full((1, _DH)),
                _full((_DH, _DH)), _full((1, _DH)),
                _full((_DH, _DH)),
                _full((1, 64)),
                _full((_DH, _DOUT)), _full((1, _DOUT)),
                _full((_DOUT, _DOUT)), _full((1, _DOUT))]
    out_specs = [pl.BlockSpec((2, _NG, _HW), lambda i: (0, 0, 0)),
                 pl.BlockSpec((_NG, _DOUT), lambda i: (0, 0))]
    out_shape = [jax.ShapeDtypeStruct((2, _NG, _HW), jnp.float32),
                 jax.ShapeDtypeStruct((_NG, _DOUT), jnp.float32)]
    return pl.pallas_call(
        _enc_g_body, grid=(1,), in_specs=in_specs, out_specs=out_specs,
        out_shape=out_shape)(
            x, p['g1_W'], p['g1_b'].reshape(1, _DH),
            p['g2_W'], p['g2_b'].reshape(1, _DH),
            p['l0gt_Wl'], _pad_one(),
            p['up_g1_W'], p['up_g1_b'].reshape(1, _DOUT),
            p['up_g2_W'], p['up_g2_b'].reshape(1, _DOUT))


def _block_diag(w, reps):
    d = w.shape[0]
    bd = jnp.zeros((_W, _W), jnp.float32)
    for k in range(reps):
        bd = bd.at[k * d:(k + 1) * d, k * d:(k + 1) * d].set(w)
    return bd


def _group_mean():
    gm = jnp.zeros((_W, _W), jnp.float32)
    for k in range(_TS):
        gm = gm.at[k * _DH:(k + 1) * _DH, k * _DH:(k + 1) * _DH].set(
            jnp.full((_DH, _DH), 1.0 / _DH))
    return gm


def _tile_flat(v):
    return jnp.concatenate(
        [jnp.tile(v, _TS), jnp.zeros((64,), jnp.float32)]).reshape(1, _W)


def _flat(pref):
    pv = pref[...]
    return jnp.concatenate([pv[0], pv[1]], axis=1)


def _combine_body(ptt, pct, pgt, base, gm, gflat, bflat, wl1bd, wr1bd, b1flat,
                  oneh, table1_ref, base1_ref):
    def agg(pref):
        v = _flat(pref)
        deg = jnp.clip(v[:, 192:193], 1.0, None)
        return v / deg

    o = agg(ptt) + agg(pct) + agg(pgt) + _flat(base)
    r = jnp.maximum(o, 0.0)
    m = r @ gm[...]
    rm = r - m
    var = (rm * rm) @ gm[...]
    h = rm * jax.lax.rsqrt(var + 1e-5) * gflat[...] + bflat[...]
    t1 = h @ wl1bd[...] + oneh[...]
    b1 = h @ wr1bd[...] + b1flat[...]
    table1_ref[0] = t1[:, :_HW]
    table1_ref[1] = t1[:, _HW:]
    base1_ref[0] = b1[:, :_HW]
    base1_ref[1] = b1[:, _HW:]


def _combine(ptt, pct, pgt, base, p):
    B = 2000
    grid = _NT // B
    hs = _halves_spec(B)
    in_specs = [hs, hs, hs, hs,
                _full((_W, _W)), _full((1, _W)), _full((1, _W)),
                _full((_W, _W)), _full((_W, _W)), _full((1, _W)),
                _full((1, _W))]
    out_specs = [hs, hs]
    out_shape = [jax.ShapeDtypeStruct((2, _NT, _HW), jnp.float32),
                 jax.ShapeDtypeStruct((2, _NT, _HW), jnp.float32)]
    oneh = jnp.zeros((1, _W), jnp.float32).at[0, 192].set(1.0)
    return pl.pallas_call(
        _combine_body, grid=(grid,), in_specs=in_specs, out_specs=out_specs,
        out_shape=out_shape)(
            ptt, pct, pgt, base,
            _group_mean(), _tile_flat(p['ln0_g']), _tile_flat(p['ln0_b']),
            _block_diag(p['l1tt_Wl'], _TS), _block_diag(p['l1tt_Wr'], _TS),
            _tile_flat(p['l1tt_b']), oneh)


def _final_body(p1, base1, ulw, ulb, out_ref):
    v = _flat(p1)
    deg = jnp.clip(v[:, 192:193], 1.0, None)
    h1 = jnp.maximum(v / deg + _flat(base1), 0.0)
    B = h1.shape[0]
    hs = jnp.concatenate([h1[:, t * _DH:(t + 1) * _DH] for t in range(_TS)],
                         axis=0)
    y = hs @ ulw[...] + ulb[...]
    for t in range(_TS):
        out_ref[:, t, :] = y[t * B:(t + 1) * B]


def _final(p1, base1, p):
    B = 2000
    grid = _NT // B
    in_specs = [_halves_spec(B), _halves_spec(B),
                _full((_DH, _DOUT)), _full((1, _DOUT))]
    out_specs = pl.BlockSpec((B, _TS, _DOUT), lambda i: (i, 0, 0))
    out_shape = jax.ShapeDtypeStruct((_NT, _TS, _DOUT), jnp.float32)
    return pl.pallas_call(
        _final_body, grid=(grid,), in_specs=in_specs, out_specs=out_specs,
        out_shape=out_shape)(
            p1, base1, p['UL_W'], p['UL_b'].reshape(1, _DOUT))


def _pad_edges(src, dst, n_src_rows):
    npad = _EPAD - _E
    src_p = jnp.concatenate(
        [src.astype(jnp.int32), (jnp.arange(npad, dtype=jnp.int32) % 64)])
    dst_p = jnp.concatenate(
        [dst.astype(jnp.int32),
         _NT + (jnp.arange(npad, dtype=jnp.int32) % 16)])
    src_cat = jnp.concatenate([src_p, src_p + n_src_rows])
    return src_cat, dst_p


def _make_sc_kernel(num_phases, staged=()):
    mesh = plsc.VectorSubcoreMesh(core_axis_name="c", subcore_axis_name="s")

    out_type = [jax.ShapeDtypeStruct((2, _ACC_ROWS, _HW), jnp.float32)
                ] * num_phases

    scratch = [pltpu.VMEM((_NCH2, _K), jnp.int32),
               pltpu.VMEM((_NCH2, _K), jnp.int32),
               pltpu.VMEM((_K, _HW), jnp.float32),
               pltpu.VMEM((_K, _HW), jnp.float32),
               pltpu.VMEM_SHARED((_ACC_ROWS, _HW), jnp.float32),
               pltpu.VMEM_SHARED((2 * _NG, _HW), jnp.float32),
               pltpu.SemaphoreType.DMA,
               pltpu.SemaphoreType.DMA]

    def body(*refs):
        n_in = 3 * num_phases + 1
        ins = refs[:n_in]
        outs = refs[n_in:n_in + num_phases]
        (src_s, dst_s, rows_a, rows_b, acc, tab_sh, sem_a,
         sem_b) = refs[n_in + num_phases:]
        zrows_hbm = ins[3 * num_phases]

        cid = lax.axis_index("c")
        sid = lax.axis_index("s")
        r0 = sid * _RPT

        def ranged(copy_fn):
            for z in range(4):
                copy_fn(r0 + z * _K, _K)

            @pl.when(sid < 15)
            def _():
                copy_fn(r0 + 4 * _K, _RPT - 4 * _K)

            @pl.when(sid == 15)
            def _():
                copy_fn(r0 + 4 * _K, _RPT_LAST - 4 * _K)

        for ph in range(num_phases):
            table = ins[3 * ph]
            src = ins[3 * ph + 1]
            dst = ins[3 * ph + 2]
            pout = outs[ph]

            if ph in staged:
                @pl.when(sid == 0)
                def _():
                    pltpu.sync_copy(table, tab_sh)
            ranged(lambda rs, n: pltpu.sync_copy(
                zrows_hbm.at[pl.ds(0, n)], acc.at[pl.ds(rs, n)]))
            plsc.subcore_barrier()

            gsrc = tab_sh if ph in staged else table

            def gstart(c, rows, sem):
                pltpu.async_copy(gsrc.at[src_s.at[c]], rows, sem)

            def gwait(rows, sem):
                pltpu.make_async_copy(gsrc.at[src_s.at[0]], rows, sem).wait()

            for hh in range(2):
                pltpu.sync_copy(
                    src.at[pl.ds(cid * (_EPAD // _K) + sid * _NCH
                                 + hh * _NCH2, _NCH2)], src_s)
                pltpu.sync_copy(
                    dst.at[pl.ds(sid * _NCH + hh * _NCH2, _NCH2)], dst_s)
                gstart(0, rows_a, sem_a)

                def pair(s, carry):
                    c0 = 2 * s
                    gstart(c0 + 1, rows_b, sem_b)
                    gwait(rows_a, sem_a)
                    pltpu.sync_copy(rows_a, acc.at[dst_s.at[c0]], add=True)

                    @pl.when(s < _NCH2 // 2 - 1)
                    def _():
                        gstart(c0 + 2, rows_a, sem_a)

                    gwait(rows_b, sem_b)
                    pltpu.sync_copy(rows_b, acc.at[dst_s.at[c0 + 1]],
                                    add=True)
                    return carry

                lax.fori_loop(0, _NCH2 // 2, pair, 0)
            plsc.subcore_barrier()

            ranged(lambda rs, n: pltpu.sync_copy(
                acc.at[pl.ds(rs, n)], pout.at[cid, pl.ds(rs, n)]))

    return pl.kernel(body, out_type=out_type, mesh=mesh,
                     scratch_types=scratch)


def _sc_seg_sums(tables_src_dst):
    num_phases = len(tables_src_dst)
    staged = tuple(i for i, (t, _, _) in enumerate(tables_src_dst)
                   if t.shape[1] == _NG)
    kern = _make_sc_kernel(num_phases, staged)
    zrows = jnp.zeros((_K, _HW), jnp.float32)
    args = []
    for table, src, dst in tables_src_dst:
        args += [table.reshape(2 * table.shape[1], _HW),
                 src.reshape(2 * _EPAD // _K, _K),
                 dst.reshape(_EPAD // _K, _K)]
    args += [zrows]
    out = kern(*args)
    return tuple(out) if isinstance(out, (list, tuple)) else (out,)


def kernel(x_t, x_c, x_g, ei_tt, ei_ct, ei_gt_src, ei_gt_dst, params):
    p = params

    table_tt, base, dec_tt = _encoder_t(x_t, p)
    table_ct, dec_tc = _encoder_c(x_c, p)
    table_gt, dec_sg = _encoder_g(x_g, p)

    s_tt, d_tt = _pad_edges(ei_tt[0], ei_tt[1], _NT)
    s_ct, d_ct = _pad_edges(ei_ct[0], ei_ct[1], _NT)
    s_gt, d_gt = _pad_edges(ei_gt_src, ei_gt_dst, _NG)

    ptt, pct, pgt = _sc_seg_sums(
        [(table_tt, s_tt, d_tt), (table_ct, s_ct, d_ct),
         (table_gt, s_gt, d_gt)])

    table1, base1 = _combine(ptt, pct, pgt, base, p)

    (p1,) = _sc_seg_sums([(table1, s_tt, d_tt)])

    x_target = _final(p1, base1, p)
    return (x_target, dec_tt, dec_tc, dec_sg)


# speedup vs baseline: 1.2143x; 1.2143x over previous
import functools

import jax
import jax.numpy as jnp
from jax import lax
from jax.experimental import pallas as pl
from jax.experimental.pallas import tpu as pltpu
from jax.experimental.pallas import tpu_sc as plsc

_NT = 10000
_NG = 100
_T = 8
_TS = 6
_FH = 2
_DH = 32
_DIN = 128
_DOUT = 128
_E = 160000
_W = 256
_HW = 128

_K = 128
_EPAD = 163840
_EPT = _EPAD // 16
_NCH = _EPT // _K
_NCH2 = _NCH // 2
_ACC_ROWS = _NT + 16
_RPT = 632
_RPT_LAST = _ACC_ROWS - 15 * _RPT


def _full(shape):
    nd = len(shape)
    return pl.BlockSpec(shape, lambda i, _nd=nd: (0,) * _nd)


def _cat_half(pieces, pad):
    if pad is not None:
        pieces = pieces + [pad]
    return jnp.concatenate(pieces, axis=1)


def _enc3_body(x_ref, w1, b1, w2, b2, w3, b3, wl, wr, bs, padb, u1w, u1b,
               u2w, u2b, table_ref, base_ref, dec_ref):
    B = x_ref.shape[0]
    x = x_ref[...].reshape(B * _T, _DIN)
    h = jnp.maximum(x @ w1[...] + b1[...], 0.0)
    h = jnp.maximum(h @ w2[...] + b2[...], 0.0)
    h = jnp.maximum(h @ w3[...] + b3[...], 0.0)
    h3 = h.reshape(B, _T, _DH)
    hk = [h3[:, k, :] for k in range(_T)]
    pad = jnp.broadcast_to(padb[...], (B, 64))
    zpad = jnp.zeros((B, 64), jnp.float32)
    wlv = wl[...]
    table_ref[0] = _cat_half([hk[k] @ wlv for k in range(4)], None)
    table_ref[1] = _cat_half([hk[4] @ wlv, hk[5] @ wlv], pad)
    if base_ref is not None:
        wrv = wr[...]
        bsv = bs[...]
        base_ref[0] = _cat_half([hk[k] @ wrv + bsv for k in range(4)], None)
        base_ref[1] = _cat_half([hk[4] @ wrv + bsv, hk[5] @ wrv + bsv], zpad)
    d = h3[:, _TS:, :].reshape(B * _FH, _DH)
    d = jnp.maximum(d @ u1w[...] + u1b[...], 0.0)
    d = jnp.maximum(d @ u2w[...] + u2b[...], 0.0)
    dec_ref[...] = d.reshape(B, _FH, _DOUT)


def _enc3_nobase_body(x_ref, w1, b1, w2, b2, w3, b3, wl, padb, u1w, u1b,
                      u2w, u2b, table_ref, dec_ref):
    _enc3_body(x_ref, w1, b1, w2, b2, w3, b3, wl, None, None, padb, u1w, u1b,
               u2w, u2b, table_ref, None, dec_ref)


def _halves_spec(B):
    return pl.BlockSpec((2, B, _HW), lambda i: (0, i, 0))


def _pad_one():
    return jnp.zeros((1, 64), jnp.float32).at[0, 0].set(1.0)


def _encoder_t(x, p):
    B = 2000
    grid = _NT // B
    in_specs = [pl.BlockSpec((B, _T, _DIN), lambda i: (i, 0, 0)),
                _full((_DIN, _DH)), _full((1, _DH)),
                _full((_DH, _DH)), _full((1, _DH)),
                _full((_DH, _DH)), _full((1, _DH)),
                _full((_DH, _DH)), _full((_DH, _DH)), _full((1, _DH)),
                _full((1, 64)),
                _full((_DH, _DOUT)), _full((1, _DOUT)),
                _full((_DOUT, _DOUT)), _full((1, _DOUT))]
    out_specs = [_halves_spec(B), _halves_spec(B),
                 pl.BlockSpec((B, _FH, _DOUT), lambda i: (i, 0, 0))]
    out_shape = [jax.ShapeDtypeStruct((2, _NT, _HW), jnp.float32),
                 jax.ShapeDtypeStruct((2, _NT, _HW), jnp.float32),
                 jax.ShapeDtypeStruct((_NT, _FH, _DOUT), jnp.float32)]
    wr_sum = p['l0tt_Wr'] + p['l0ct_Wr'] + p['l0gt_Wr']
    b_sum = (p['l0tt_b'] + p['l0ct_b'] + p['l0gt_b']).reshape(1, _DH)
    return pl.pallas_call(
        _enc3_body, grid=(grid,), in_specs=in_specs, out_specs=out_specs,
        out_shape=out_shape)(
            x, p['t1_W'], p['t1_b'].reshape(1, _DH),
            p['t2_W'], p['t2_b'].reshape(1, _DH),
            p['t3_W'], p['t3_b'].reshape(1, _DH),
            p['l0tt_Wl'], wr_sum, b_sum, _pad_one(),
            p['up_t1_W'], p['up_t1_b'].reshape(1, _DOUT),
            p['up_t2_W'], p['up_t2_b'].reshape(1, _DOUT))


def _encoder_c(x, p):
    B = 2000
    grid = _NT // B
    in_specs = [pl.BlockSpec((B, _T, _DIN), lambda i: (i, 0, 0)),
                _full((_DIN, _DH)), _full((1, _DH)),
                _full((_DH, _DH)), _full((1, _DH)),
                _full((_DH, _DH)), _full((1, _DH)),
                _full((_DH, _DH)),
                _full((1, 64)),
                _full((_DH, _DOUT)), _full((1, _DOUT)),
                _full((_DOUT, _DOUT)), _full((1, _DOUT))]
    out_specs = [_halves_spec(B),
                 pl.BlockSpec((B, _FH, _DOUT), lambda i: (i, 0, 0))]
    out_shape = [jax.ShapeDtypeStruct((2, _NT, _HW), jnp.float32),
                 jax.ShapeDtypeStruct((_NT, _FH, _DOUT), jnp.float32)]
    return pl.pallas_call(
        _enc3_nobase_body, grid=(grid,), in_specs=in_specs,
        out_specs=out_specs, out_shape=out_shape)(
            x, p['c1_W'], p['c1_b'].reshape(1, _DH),
            p['c2_W'], p['c2_b'].reshape(1, _DH),
            p['c3_W'], p['c3_b'].reshape(1, _DH),
            p['l0ct_Wl'], _pad_one(),
            p['up_c1_W'], p['up_c1_b'].reshape(1, _DOUT),
            p['up_c2_W'], p['up_c2_b'].reshape(1, _DOUT))


def _enc_g_body(x_ref, w1, b1, w2, b2, wl, padb, u1w, u1b, u2w, u2b,
                table_ref, dec_ref):
    B = x_ref.shape[0]
    x = x_ref[...].reshape(B * _T, _DIN)
    h = jnp.maximum(x @ w1[...] + b1[...], 0.0)
    h = jnp.maximum(h @ w2[...] + b2[...], 0.0)
    h3 = h.reshape(B, _T, _DH)
    hk = [h3[:, k, :] for k in range(_T)]
    pad = jnp.broadcast_to(padb[...], (B, 64))
    wlv = wl[...]
    table_ref[0] = _cat_half([hk[k] @ wlv for k in range(4)], None)
    table_ref[1] = _cat_half([hk[4] @ wlv, hk[5] @ wlv], pad)
    d = hk[_T - 1]
    d = jnp.maximum(d @ u1w[...] + u1b[...], 0.0)
    d = jnp.maximum(d @ u2w[...] + u2b[...], 0.0)
    dec_ref[...] = d


def _encoder_g(x, p):
    in_specs = [pl.BlockSpec((_NG, _T, _DIN), lambda i: (0, 0, 0)),
                _full((_DIN, _DH)), _full((1, _DH)),
                _full((_DH, _DH)), _full((1, _DH)),
                _full((_DH, _DH)),
                _full((1, 64)),
                _full((_DH, _DOUT)), _full((1, _DOUT)),
                _full((_DOUT, _DOUT)), _full((1, _DOUT))]
    out_specs = [pl.BlockSpec((2, _NG, _HW), lambda i: (0, 0, 0)),
                 pl.BlockSpec((_NG, _DOUT), lambda i: (0, 0))]
    out_shape = [jax.ShapeDtypeStruct((2, _NG, _HW), jnp.float32),
                 jax.ShapeDtypeStruct((_NG, _DOUT), jnp.float32)]
    return pl.pallas_call(
        _enc_g_body, grid=(1,), in_specs=in_specs, out_specs=out_specs,
        out_shape=out_shape)(
            x, p['g1_W'], p['g1_b'].reshape(1, _DH),
            p['g2_W'], p['g2_b'].reshape(1, _DH),
            p['l0gt_Wl'], _pad_one(),
            p['up_g1_W'], p['up_g1_b'].reshape(1, _DOUT),
            p['up_g2_W'], p['up_g2_b'].reshape(1, _DOUT))


def _block_diag(w, reps):
    d = w.shape[0]
    bd = jnp.zeros((_W, _W), jnp.float32)
    for k in range(reps):
        bd = bd.at[k * d:(k + 1) * d, k * d:(k + 1) * d].set(w)
    return bd


def _group_mean():
    gm = jnp.zeros((_W, _W), jnp.float32)
    for k in range(_TS):
        gm = gm.at[k * _DH:(k + 1) * _DH, k * _DH:(k + 1) * _DH].set(
            jnp.full((_DH, _DH), 1.0 / _DH))
    return gm


def _tile_flat(v):
    return jnp.concatenate(
        [jnp.tile(v, _TS), jnp.zeros((64,), jnp.float32)]).reshape(1, _W)


def _flat(pref):
    pv = pref[...]
    return jnp.concatenate([pv[0], pv[1]], axis=1)


def _combine_body(ptt, pct, pgt, base, gm, gflat, bflat, wl1bd, wr1bd, b1flat,
                  oneh, table1_ref, base1_ref):
    def agg(pref):
        v = _flat(pref)
        deg = jnp.clip(v[:, 192:193], 1.0, None)
        return v / deg

    o = agg(ptt) + agg(pct) + agg(pgt) + _flat(base)
    r = jnp.maximum(o, 0.0)
    m = r @ gm[...]
    rm = r - m
    var = (rm * rm) @ gm[...]
    h = rm * jax.lax.rsqrt(var + 1e-5) * gflat[...] + bflat[...]
    t1 = h @ wl1bd[...] + oneh[...]
    b1 = h @ wr1bd[...] + b1flat[...]
    table1_ref[0] = t1[:, :_HW]
    table1_ref[1] = t1[:, _HW:]
    base1_ref[0] = b1[:, :_HW]
    base1_ref[1] = b1[:, _HW:]


def _combine(ptt, pct, pgt, base, p):
    B = 2000
    grid = _NT // B
    hs = _halves_spec(B)
    in_specs = [hs, hs, hs, hs,
                _full((_W, _W)), _full((1, _W)), _full((1, _W)),
                _full((_W, _W)), _full((_W, _W)), _full((1, _W)),
                _full((1, _W))]
    out_specs = [hs, hs]
    out_shape = [jax.ShapeDtypeStruct((2, _NT, _HW), jnp.float32),
                 jax.ShapeDtypeStruct((2, _NT, _HW), jnp.float32)]
    oneh = jnp.zeros((1, _W), jnp.float32).at[0, 192].set(1.0)
    return pl.pallas_call(
        _combine_body, grid=(grid,), in_specs=in_specs, out_specs=out_specs,
        out_shape=out_shape)(
            ptt, pct, pgt, base,
            _group_mean(), _tile_flat(p['ln0_g']), _tile_flat(p['ln0_b']),
            _block_diag(p['l1tt_Wl'], _TS), _block_diag(p['l1tt_Wr'], _TS),
            _tile_flat(p['l1tt_b']), oneh)


def _final_body(p1, base1, ulw, ulb, out_ref):
    v = _flat(p1)
    deg = jnp.clip(v[:, 192:193], 1.0, None)
    h1 = jnp.maximum(v / deg + _flat(base1), 0.0)
    ulwv = ulw[...]
    ulbv = ulb[...]
    for t in range(_TS):
        out_ref[:, t, :] = h1[:, t * _DH:(t + 1) * _DH] @ ulwv + ulbv


def _final(p1, base1, p):
    B = 2000
    grid = _NT // B
    in_specs = [_halves_spec(B), _halves_spec(B),
                _full((_DH, _DOUT)), _full((1, _DOUT))]
    out_specs = pl.BlockSpec((B, _TS, _DOUT), lambda i: (i, 0, 0))
    out_shape = jax.ShapeDtypeStruct((_NT, _TS, _DOUT), jnp.float32)
    return pl.pallas_call(
        _final_body, grid=(grid,), in_specs=in_specs, out_specs=out_specs,
        out_shape=out_shape)(
            p1, base1, p['UL_W'], p['UL_b'].reshape(1, _DOUT))


def _pad_edges(src, dst, n_src_rows):
    npad = _EPAD - _E
    src_p = jnp.concatenate(
        [src.astype(jnp.int32), (jnp.arange(npad, dtype=jnp.int32) % 64)])
    dst_p = jnp.concatenate(
        [dst.astype(jnp.int32),
         _NT + (jnp.arange(npad, dtype=jnp.int32) % 16)])
    src_cat = jnp.concatenate([src_p, src_p + n_src_rows])
    return src_cat, dst_p


def _make_sc_kernel(num_phases, staged=()):
    mesh = plsc.VectorSubcoreMesh(core_axis_name="c", subcore_axis_name="s")

    out_type = [jax.ShapeDtypeStruct((2, _ACC_ROWS, _HW), jnp.float32)
                ] * num_phases

    scratch = [pltpu.VMEM((_NCH2, _K), jnp.int32),
               pltpu.VMEM((_NCH2, _K), jnp.int32),
               pltpu.VMEM((_K, _HW), jnp.float32),
               pltpu.VMEM((_K, _HW), jnp.float32),
               pltpu.VMEM_SHARED((_ACC_ROWS, _HW), jnp.float32),
               pltpu.VMEM_SHARED((2 * _NG, _HW), jnp.float32),
               pltpu.SemaphoreType.DMA,
               pltpu.SemaphoreType.DMA]

    def body(*refs):
        n_in = 3 * num_phases + 1
        ins = refs[:n_in]
        outs = refs[n_in:n_in + num_phases]
        (src_s, dst_s, rows_a, rows_b, acc, tab_sh, sem_a,
         sem_b) = refs[n_in + num_phases:]
        zrows_hbm = ins[3 * num_phases]

        cid = lax.axis_index("c")
        sid = lax.axis_index("s")
        r0 = sid * _RPT

        def ranged(copy_fn):
            for z in range(4):
                copy_fn(r0 + z * _K, _K)

            @pl.when(sid < 15)
            def _():
                copy_fn(r0 + 4 * _K, _RPT - 4 * _K)

            @pl.when(sid == 15)
            def _():
                copy_fn(r0 + 4 * _K, _RPT_LAST - 4 * _K)

        for ph in range(num_phases):
            table = ins[3 * ph]
            src = ins[3 * ph + 1]
            dst = ins[3 * ph + 2]
            pout = outs[ph]

            if ph in staged:
                @pl.when(sid == 0)
                def _():
                    pltpu.sync_copy(table, tab_sh)
            ranged(lambda rs, n: pltpu.sync_copy(
                zrows_hbm.at[pl.ds(0, n)], acc.at[pl.ds(rs, n)]))
            plsc.subcore_barrier()

            gsrc = tab_sh if ph in staged else table

            def gstart(c, rows, sem):
                pltpu.async_copy(gsrc.at[src_s.at[c]], rows, sem)

            def gwait(rows, sem):
                pltpu.make_async_copy(gsrc.at[src_s.at[0]], rows, sem).wait()

            for hh in range(2):
                pltpu.sync_copy(
                    src.at[pl.ds(cid * (_EPAD // _K) + sid * _NCH
                                 + hh * _NCH2, _NCH2)], src_s)
                pltpu.sync_copy(
                    dst.at[pl.ds(sid * _NCH + hh * _NCH2, _NCH2)], dst_s)
                gstart(0, rows_a, sem_a)

                def pair(s, carry):
                    c0 = 2 * s
                    gstart(c0 + 1, rows_b, sem_b)
                    gwait(rows_a, sem_a)
                    pltpu.sync_copy(rows_a, acc.at[dst_s.at[c0]], add=True)

                    @pl.when(s < _NCH2 // 2 - 1)
                    def _():
                        gstart(c0 + 2, rows_a, sem_a)

                    gwait(rows_b, sem_b)
                    pltpu.sync_copy(rows_b, acc.at[dst_s.at[c0 + 1]],
                                    add=True)
                    return carry

                lax.fori_loop(0, _NCH2 // 2, pair, 0)
            plsc.subcore_barrier()

            ranged(lambda rs, n: pltpu.sync_copy(
                acc.at[pl.ds(rs, n)], pout.at[cid, pl.ds(rs, n)]))

    return pl.kernel(body, out_type=out_type, mesh=mesh,
                     scratch_types=scratch)


def _sc_seg_sums(tables_src_dst):
    num_phases = len(tables_src_dst)
    staged = tuple(i for i, (t, _, _) in enumerate(tables_src_dst)
                   if t.shape[1] == _NG)
    kern = _make_sc_kernel(num_phases, staged)
    zrows = jnp.zeros((_K, _HW), jnp.float32)
    args = []
    for table, src, dst in tables_src_dst:
        args += [table.reshape(2 * table.shape[1], _HW),
                 src.reshape(2 * _EPAD // _K, _K),
                 dst.reshape(_EPAD // _K, _K)]
    args += [zrows]
    out = kern(*args)
    return tuple(out) if isinstance(out, (list, tuple)) else (out,)


def kernel(x_t, x_c, x_g, ei_tt, ei_ct, ei_gt_src, ei_gt_dst, params):
    p = params

    table_tt, base, dec_tt = _encoder_t(x_t, p)
    table_ct, dec_tc = _encoder_c(x_c, p)
    table_gt, dec_sg = _encoder_g(x_g, p)

    s_tt, d_tt = _pad_edges(ei_tt[0], ei_tt[1], _NT)
    s_ct, d_ct = _pad_edges(ei_ct[0], ei_ct[1], _NT)
    s_gt, d_gt = _pad_edges(ei_gt_src, ei_gt_dst, _NG)

    (ptt,) = _sc_seg_sums([(table_tt, s_tt, d_tt)])
    (pct,) = _sc_seg_sums([(table_ct, s_ct, d_ct)])
    (pgt,) = _sc_seg_sums([(table_gt, s_gt, d_gt)])

    table1, base1 = _combine(ptt, pct, pgt, base, p)

    (p1,) = _sc_seg_sums([(table1, s_tt, d_tt)])

    x_target = _final(p1, base1, p)
    return (x_target, dec_tt, dec_tc, dec_sg)


# speedup vs baseline: 1.2146x; 1.0002x over previous
import functools

import jax
import jax.numpy as jnp
from jax import lax
from jax.experimental import pallas as pl
from jax.experimental.pallas import tpu as pltpu
from jax.experimental.pallas import tpu_sc as plsc

_NT = 10000
_NG = 100
_T = 8
_TS = 6
_FH = 2
_DH = 32
_DIN = 128
_DOUT = 128
_E = 160000
_W = 256
_HW = 128

_K = 128
_EPAD = 163840
_EPT = _EPAD // 16
_NCH = _EPT // _K
_NCH2 = _NCH // 2
_ACC_ROWS = _NT + 16
_RPT = 632
_RPT_LAST = _ACC_ROWS - 15 * _RPT


def _full(shape):
    nd = len(shape)
    return pl.BlockSpec(shape, lambda i, _nd=nd: (0,) * _nd)


def _cat_half(pieces, pad):
    if pad is not None:
        pieces = pieces + [pad]
    return jnp.concatenate(pieces, axis=1)


def _enc3_body(x_ref, w1, b1, w2, b2, w3, b3, wl, wr, bs, padb, u1w, u1b,
               u2w, u2b, table_ref, base_ref, dec_ref):
    B = x_ref.shape[0]
    x = x_ref[...].reshape(B * _T, _DIN)
    h = jnp.maximum(x @ w1[...] + b1[...], 0.0)
    h = jnp.maximum(h @ w2[...] + b2[...], 0.0)
    h = jnp.maximum(h @ w3[...] + b3[...], 0.0)
    h3 = h.reshape(B, _T, _DH)
    hk = [h3[:, k, :] for k in range(_T)]
    pad = jnp.broadcast_to(padb[...], (B, 64))
    zpad = jnp.zeros((B, 64), jnp.float32)
    wlv = wl[...]
    table_ref[0] = _cat_half([hk[k] @ wlv for k in range(4)], None)
    table_ref[1] = _cat_half([hk[4] @ wlv, hk[5] @ wlv], pad)
    if base_ref is not None:
        wrv = wr[...]
        bsv = bs[...]
        base_ref[0] = _cat_half([hk[k] @ wrv + bsv for k in range(4)], None)
        base_ref[1] = _cat_half([hk[4] @ wrv + bsv, hk[5] @ wrv + bsv], zpad)
    d = h3[:, _TS:, :].reshape(B * _FH, _DH)
    d = jnp.maximum(d @ u1w[...] + u1b[...], 0.0)
    d = jnp.maximum(d @ u2w[...] + u2b[...], 0.0)
    dec_ref[...] = d.reshape(B, _FH, _DOUT)


def _enc3_nobase_body(x_ref, w1, b1, w2, b2, w3, b3, wl, padb, u1w, u1b,
                      u2w, u2b, table_ref, dec_ref):
    _enc3_body(x_ref, w1, b1, w2, b2, w3, b3, wl, None, None, padb, u1w, u1b,
               u2w, u2b, table_ref, None, dec_ref)


def _halves_spec(B):
    return pl.BlockSpec((2, B, _HW), lambda i: (0, i, 0))


def _pad_one():
    return jnp.zeros((1, 64), jnp.float32).at[0, 0].set(1.0)


def _encoder_t(x, p):
    B = 2000
    grid = _NT // B
    in_specs = [pl.BlockSpec((B, _T, _DIN), lambda i: (i, 0, 0)),
                _full((_DIN, _DH)), _full((1, _DH)),
                _full((_DH, _DH)), _full((1, _DH)),
                _full((_DH, _DH)), _full((1, _DH)),
                _full((_DH, _DH)), _full((_DH, _DH)), _full((1, _DH)),
                _full((1, 64)),
                _full((_DH, _DOUT)), _full((1, _DOUT)),
                _full((_DOUT, _DOUT)), _full((1, _DOUT))]
    out_specs = [_halves_spec(B), _halves_spec(B),
                 pl.BlockSpec((B, _FH, _DOUT), lambda i: (i, 0, 0))]
    out_shape = [jax.ShapeDtypeStruct((2, _NT, _HW), jnp.float32),
                 jax.ShapeDtypeStruct((2, _NT, _HW), jnp.float32),
                 jax.ShapeDtypeStruct((_NT, _FH, _DOUT), jnp.float32)]
    wr_sum = p['l0tt_Wr'] + p['l0ct_Wr'] + p['l0gt_Wr']
    b_sum = (p['l0tt_b'] + p['l0ct_b'] + p['l0gt_b']).reshape(1, _DH)
    return pl.pallas_call(
        _enc3_body, grid=(grid,), in_specs=in_specs, out_specs=out_specs,
        out_shape=out_shape)(
            x, p['t1_W'], p['t1_b'].reshape(1, _DH),
            p['t2_W'], p['t2_b'].reshape(1, _DH),
            p['t3_W'], p['t3_b'].reshape(1, _DH),
            p['l0tt_Wl'], wr_sum, b_sum, _pad_one(),
            p['up_t1_W'], p['up_t1_b'].reshape(1, _DOUT),
            p['up_t2_W'], p['up_t2_b'].reshape(1, _DOUT))


def _encoder_c(x, p):
    B = 2000
    grid = _NT // B
    in_specs = [pl.BlockSpec((B, _T, _DIN), lambda i: (i, 0, 0)),
                _full((_DIN, _DH)), _full((1, _DH)),
                _full((_DH, _DH)), _full((1, _DH)),
                _full((_DH, _DH)), _full((1, _DH)),
                _full((_DH, _DH)),
                _full((1, 64)),
                _full((_DH, _DOUT)), _full((1, _DOUT)),
                _full((_DOUT, _DOUT)), _full((1, _DOUT))]
    out_specs = [_halves_spec(B),
                 pl.BlockSpec((B, _FH, _DOUT), lambda i: (i, 0, 0))]
    out_shape = [jax.ShapeDtypeStruct((2, _NT, _HW), jnp.float32),
                 jax.ShapeDtypeStruct((_NT, _FH, _DOUT), jnp.float32)]
    return pl.pallas_call(
        _enc3_nobase_body, grid=(grid,), in_specs=in_specs,
        out_specs=out_specs, out_shape=out_shape)(
            x, p['c1_W'], p['c1_b'].reshape(1, _DH),
            p['c2_W'], p['c2_b'].reshape(1, _DH),
            p['c3_W'], p['c3_b'].reshape(1, _DH),
            p['l0ct_Wl'], _pad_one(),
            p['up_c1_W'], p['up_c1_b'].reshape(1, _DOUT),
            p['up_c2_W'], p['up_c2_b'].reshape(1, _DOUT))


def _enc_g_body(x_ref, w1, b1, w2, b2, wl, padb, u1w, u1b, u2w, u2b,
                table_ref, dec_ref):
    B = x_ref.shape[0]
    x = x_ref[...].reshape(B * _T, _DIN)
    h = jnp.maximum(x @ w1[...] + b1[...], 0.0)
    h = jnp.maximum(h @ w2[...] + b2[...], 0.0)
    h3 = h.reshape(B, _T, _DH)
    hk = [h3[:, k, :] for k in range(_T)]
    pad = jnp.broadcast_to(padb[...], (B, 64))
    wlv = wl[...]
    table_ref[0] = _cat_half([hk[k] @ wlv for k in range(4)], None)
    table_ref[1] = _cat_half([hk[4] @ wlv, hk[5] @ wlv], pad)
    d = hk[_T - 1]
    d = jnp.maximum(d @ u1w[...] + u1b[...], 0.0)
    d = jnp.maximum(d @ u2w[...] + u2b[...], 0.0)
    dec_ref[...] = d


def _encoder_g(x, p):
    in_specs = [pl.BlockSpec((_NG, _T, _DIN), lambda i: (0, 0, 0)),
                _full((_DIN, _DH)), _full((1, _DH)),
                _full((_DH, _DH)), _full((1, _DH)),
                _full((_DH, _DH)),
                _full((1, 64)),
                _full((_DH, _DOUT)), _full((1, _DOUT)),
                _full((_DOUT, _DOUT)), _full((1, _DOUT))]
    out_specs = [pl.BlockSpec((2, _NG, _HW), lambda i: (0, 0, 0)),
                 pl.BlockSpec((_NG, _DOUT), lambda i: (0, 0))]
    out_shape = [jax.ShapeDtypeStruct((2, _NG, _HW), jnp.float32),
                 jax.ShapeDtypeStruct((_NG, _DOUT), jnp.float32)]
    return pl.pallas_call(
        _enc_g_body, grid=(1,), in_specs=in_specs, out_specs=out_specs,
        out_shape=out_shape)(
            x, p['g1_W'], p['g1_b'].reshape(1, _DH),
            p['g2_W'], p['g2_b'].reshape(1, _DH),
            p['l0gt_Wl'], _pad_one(),
            p['up_g1_W'], p['up_g1_b'].reshape(1, _DOUT),
            p['up_g2_W'], p['up_g2_b'].reshape(1, _DOUT))


def _block_diag(w, reps):
    d = w.shape[0]
    bd = jnp.zeros((_W, _W), jnp.float32)
    for k in range(reps):
        bd = bd.at[k * d:(k + 1) * d, k * d:(k + 1) * d].set(w)
    return bd


def _group_mean():
    gm = jnp.zeros((_W, _W), jnp.float32)
    for k in range(_TS):
        gm = gm.at[k * _DH:(k + 1) * _DH, k * _DH:(k + 1) * _DH].set(
            jnp.full((_DH, _DH), 1.0 / _DH))
    return gm


def _tile_flat(v):
    return jnp.concatenate(
        [jnp.tile(v, _TS), jnp.zeros((64,), jnp.float32)]).reshape(1, _W)


def _flat(pref):
    pv = pref[...]
    return jnp.concatenate([pv[0], pv[1]], axis=1)


def _combine_body(ptt, pct, pgt, base, gm, gflat, bflat, wl1bd, wr1bd, b1flat,
                  oneh, table1_ref, base1_ref):
    def agg(pref):
        v = _flat(pref)
        deg = jnp.clip(v[:, 192:193], 1.0, None)
        return v / deg

    o = agg(ptt) + agg(pct) + agg(pgt) + _flat(base)
    r = jnp.maximum(o, 0.0)
    m = r @ gm[...]
    rm = r - m
    var = (rm * rm) @ gm[...]
    h = rm * jax.lax.rsqrt(var + 1e-5) * gflat[...] + bflat[...]
    t1 = h @ wl1bd[...] + oneh[...]
    b1 = h @ wr1bd[...] + b1flat[...]
    table1_ref[0] = t1[:, :_HW]
    table1_ref[1] = t1[:, _HW:]
    base1_ref[0] = b1[:, :_HW]
    base1_ref[1] = b1[:, _HW:]


def _combine(ptt, pct, pgt, base, p):
    B = 2000
    grid = _NT // B
    hs = _halves_spec(B)
    in_specs = [hs, hs, hs, hs,
                _full((_W, _W)), _full((1, _W)), _full((1, _W)),
                _full((_W, _W)), _full((_W, _W)), _full((1, _W)),
                _full((1, _W))]
    out_specs = [hs, hs]
    out_shape = [jax.ShapeDtypeStruct((2, _NT, _HW), jnp.float32),
                 jax.ShapeDtypeStruct((2, _NT, _HW), jnp.float32)]
    oneh = jnp.zeros((1, _W), jnp.float32).at[0, 192].set(1.0)
    return pl.pallas_call(
        _combine_body, grid=(grid,), in_specs=in_specs, out_specs=out_specs,
        out_shape=out_shape)(
            ptt, pct, pgt, base,
            _group_mean(), _tile_flat(p['ln0_g']), _tile_flat(p['ln0_b']),
            _block_diag(p['l1tt_Wl'], _TS), _block_diag(p['l1tt_Wr'], _TS),
            _tile_flat(p['l1tt_b']), oneh)


def _final_body(p1, base1, ulw, ulb, out_ref):
    v = _flat(p1)
    deg = jnp.clip(v[:, 192:193], 1.0, None)
    h1 = jnp.maximum(v / deg + _flat(base1), 0.0)
    ulwv = ulw[...]
    ulbv = ulb[...]
    for t in range(_TS):
        out_ref[:, t, :] = h1[:, t * _DH:(t + 1) * _DH] @ ulwv + ulbv


def _final(p1, base1, p):
    B = 2000
    grid = _NT // B
    in_specs = [_halves_spec(B), _halves_spec(B),
                _full((_DH, _DOUT)), _full((1, _DOUT))]
    out_specs = pl.BlockSpec((B, _TS, _DOUT), lambda i: (i, 0, 0))
    out_shape = jax.ShapeDtypeStruct((_NT, _TS, _DOUT), jnp.float32)
    return pl.pallas_call(
        _final_body, grid=(grid,), in_specs=in_specs, out_specs=out_specs,
        out_shape=out_shape)(
            p1, base1, p['UL_W'], p['UL_b'].reshape(1, _DOUT))


def _pad_edges(src, dst, n_src_rows):
    npad = _EPAD - _E
    src_p = jnp.concatenate(
        [src.astype(jnp.int32), (jnp.arange(npad, dtype=jnp.int32) % 64)])
    dst_p = jnp.concatenate(
        [dst.astype(jnp.int32),
         _NT + (jnp.arange(npad, dtype=jnp.int32) % 16)])
    src_cat = jnp.concatenate([src_p, src_p + n_src_rows])
    return src_cat, dst_p


def _make_sc_kernel(num_phases, staged=()):
    mesh = plsc.VectorSubcoreMesh(core_axis_name="c", subcore_axis_name="s")

    out_type = [jax.ShapeDtypeStruct((2, _ACC_ROWS, _HW), jnp.float32)
                ] * num_phases

    scratch = [pltpu.VMEM((_NCH2, _K), jnp.int32),
               pltpu.VMEM((_NCH2, _K), jnp.int32),
               pltpu.VMEM((_K, _HW), jnp.float32),
               pltpu.VMEM((_K, _HW), jnp.float32),
               pltpu.VMEM_SHARED((_ACC_ROWS, _HW), jnp.float32),
               pltpu.VMEM_SHARED((2 * _NG, _HW), jnp.float32),
               pltpu.SemaphoreType.DMA,
               pltpu.SemaphoreType.DMA]

    def body(*refs):
        n_in = 3 * num_phases + 1
        ins = refs[:n_in]
        outs = refs[n_in:n_in + num_phases]
        (src_s, dst_s, rows_a, rows_b, acc, tab_sh, sem_a,
         sem_b) = refs[n_in + num_phases:]
        zrows_hbm = ins[3 * num_phases]

        cid = lax.axis_index("c")
        sid = lax.axis_index("s")
        r0 = sid * _RPT

        def ranged(copy_fn):
            for z in range(4):
                copy_fn(r0 + z * _K, _K)

            @pl.when(sid < 15)
            def _():
                copy_fn(r0 + 4 * _K, _RPT - 4 * _K)

            @pl.when(sid == 15)
            def _():
                copy_fn(r0 + 4 * _K, _RPT_LAST - 4 * _K)

        for ph in range(num_phases):
            table = ins[3 * ph]
            src = ins[3 * ph + 1]
            dst = ins[3 * ph + 2]
            pout = outs[ph]

            if ph in staged:
                @pl.when(sid == 0)
                def _():
                    pltpu.sync_copy(table, tab_sh)
            ranged(lambda rs, n: pltpu.sync_copy(
                zrows_hbm.at[pl.ds(0, n)], acc.at[pl.ds(rs, n)]))
            plsc.subcore_barrier()

            gsrc = tab_sh if ph in staged else table

            def gstart(c, rows, sem):
                pltpu.async_copy(gsrc.at[src_s.at[c]], rows, sem)

            def gwait(rows, sem):
                pltpu.make_async_copy(gsrc.at[src_s.at[0]], rows, sem).wait()

            for hh in range(2):
                pltpu.sync_copy(
                    src.at[pl.ds(cid * (_EPAD // _K) + sid * _NCH
                                 + hh * _NCH2, _NCH2)], src_s)
                pltpu.sync_copy(
                    dst.at[pl.ds(sid * _NCH + hh * _NCH2, _NCH2)], dst_s)
                gstart(0, rows_a, sem_a)

                def pair(s, carry):
                    c0 = 2 * s
                    gstart(c0 + 1, rows_b, sem_b)
                    gwait(rows_a, sem_a)
                    pltpu.sync_copy(rows_a, acc.at[dst_s.at[c0]], add=True)

                    @pl.when(s < _NCH2 // 2 - 1)
                    def _():
                        gstart(c0 + 2, rows_a, sem_a)

                    gwait(rows_b, sem_b)
                    pltpu.sync_copy(rows_b, acc.at[dst_s.at[c0 + 1]],
                                    add=True)
                    return carry

                lax.fori_loop(0, _NCH2 // 2, pair, 0)
            plsc.subcore_barrier()

            ranged(lambda rs, n: pltpu.sync_copy(
                acc.at[pl.ds(rs, n)], pout.at[cid, pl.ds(rs, n)]))

    return pl.kernel(body, out_type=out_type, mesh=mesh,
                     scratch_types=scratch)


def _sc_seg_sums(tables_src_dst):
    num_phases = len(tables_src_dst)
    staged = tuple(i for i, (t, _, _) in enumerate(tables_src_dst)
                   if t.shape[1] == _NG)
    kern = _make_sc_kernel(num_phases, staged)
    zrows = jnp.zeros((_K, _HW), jnp.float32)
    args = []
    for table, src, dst in tables_src_dst:
        args += [table.reshape(2 * table.shape[1], _HW),
                 src.reshape(2 * _EPAD // _K, _K),
                 dst.reshape(_EPAD // _K, _K)]
    args += [zrows]
    out = kern(*args)
    return tuple(out) if isinstance(out, (list, tuple)) else (out,)


def kernel(x_t, x_c, x_g, ei_tt, ei_ct, ei_gt_src, ei_gt_dst, params):
    p = params

    s_tt, d_tt = _pad_edges(ei_tt[0], ei_tt[1], _NT)
    s_ct, d_ct = _pad_edges(ei_ct[0], ei_ct[1], _NT)
    s_gt, d_gt = _pad_edges(ei_gt_src, ei_gt_dst, _NG)

    table_gt, dec_sg = _encoder_g(x_g, p)
    (pgt,) = _sc_seg_sums([(table_gt, s_gt, d_gt)])
    table_ct, dec_tc = _encoder_c(x_c, p)
    (pct,) = _sc_seg_sums([(table_ct, s_ct, d_ct)])
    table_tt, base, dec_tt = _encoder_t(x_t, p)
    (ptt,) = _sc_seg_sums([(table_tt, s_tt, d_tt)])

    table1, base1 = _combine(ptt, pct, pgt, base, p)

    (p1,) = _sc_seg_sums([(table1, s_tt, d_tt)])

    x_target = _final(p1, base1, p)
    return (x_target, dec_tt, dec_tc, dec_sg)
